# baseline (speedup 1.0000x reference)
import functools

import jax
import jax.numpy as jnp
from jax import lax
from jax.experimental import pallas as pl
from jax.experimental.pallas import tpu as pltpu
from jax.experimental.pallas import tpu_sc as plsc

N = 10000
E = 160000
D = 256
NC = 2
NS = 16
L = 16

HALF = N // 2
SLABR = 5120
FR = SLABR // NS
ADJ = SLABR - HALF
DUMP = HALF
TBL = NC * SLABR

EC = E // NS
BLK = 128
NB = 80
ECP = NB * BLK

_mesh = plsc.VectorSubcoreMesh(
    core_axis_name="c", subcore_axis_name="s", num_cores=NC, num_subcores=NS
)


def _build_dst_idx(dstbuf, idx2d, c):
    base = c * HALF
    lane = lax.iota(jnp.int32, L)

    def body(i, _):
        j = i // (BLK // L)
        l = i % (BLK // L)
        v = dstbuf[pl.ds(i * L, L)]
        loc = v - base
        ok = (loc >= 0) & (loc < HALF)
        dump = DUMP + (i % 7) * L + lane
        idx2d[j, pl.ds(l * L, L)] = jnp.where(ok, loc, dump)
        return 0

    lax.fori_loop(0, ECP // L, body, 0)


def _load_edge_chunk(ehbm, ebuf, s, pad_value):
    pltpu.sync_copy(ehbm.at[pl.ds(s * EC, EC)], ebuf.at[pl.ds(0, EC)])
    pad = jnp.full((L,), pad_value, jnp.int32)

    def body(i, _):
        ebuf[pl.ds(EC + i * L, L)] = pad
        return 0

    lax.fori_loop(0, (ECP - EC) // L, body, 0)


def _deg_body(dst_hbm, deg_hbm, dstbuf, idx2d, ones, zb, acc):
    c = lax.axis_index("c")
    s = lax.axis_index("s")

    z16 = jnp.zeros((L,), jnp.float32)
    o16 = jnp.ones((L,), jnp.float32)

    def fill(i, _):
        zb[pl.ds(i * L, L)] = z16
        return 0

    lax.fori_loop(0, FR // L, fill, 0)

    def fill1(i, _):
        ones[pl.ds(i * L, L)] = o16
        return 0

    lax.fori_loop(0, BLK // L, fill1, 0)

    _load_edge_chunk(dst_hbm, dstbuf, s, -1)
    _build_dst_idx(dstbuf, idx2d, c)

    pltpu.sync_copy(zb, acc.at[pl.ds(s * FR, FR)])
    plsc.subcore_barrier()

    def scat(j, _):
        pltpu.sync_copy(ones, acc.at[idx2d.at[j]], add=True)
        return 0

    lax.fori_loop(0, NB, scat, 0)
    plsc.subcore_barrier()
    pltpu.sync_copy(acc.at[pl.ds(s * FR, FR)], zb)
    pltpu.sync_copy(zb, deg_hbm.at[pl.ds(c * SLABR + s * FR, FR)])


_deg_call = functools.partial(
    pl.kernel,
    out_type=jax.ShapeDtypeStruct((TBL,), jnp.float32),
    mesh=_mesh,
    scratch_types=[
        pltpu.VMEM((ECP,), jnp.int32),
        pltpu.VMEM((NB, BLK), jnp.int32),
        pltpu.VMEM((BLK,), jnp.float32),
        pltpu.VMEM((FR,), jnp.float32),
        pltpu.VMEM_SHARED((SLABR,), jnp.float32),
    ],
)(_deg_body)


DH = D // 2


def _agg_body(src_hbm, dst_hbm, tbl_hbm, out_hbm, srcraw, dstb, srcx,
              idx2d, rbuf, acc, gsem0, gsem1):
    c = lax.axis_index("c")
    s = lax.axis_index("s")

    z16 = jnp.zeros((L,), jnp.float32)

    _load_edge_chunk(src_hbm, srcraw, s, 0)

    def adjust(i, _):
        v = srcraw[pl.ds(i * L, L)]
        srcraw[pl.ds(i * L, L)] = jnp.where(v >= HALF, v + ADJ, v)
        return 0

    lax.fori_loop(0, ECP // L, adjust, 0)

    _load_edge_chunk(dst_hbm, dstb, s, -1)
    _build_dst_idx(dstb, idx2d, c)

    npairs = NB // 2

    for h in range(2):
        def half_idx(i, _):
            srcx[pl.ds(i * L, L)] = 2 * srcraw[pl.ds(i * L, L)] + h
            return 0

        lax.fori_loop(0, ECP // L, half_idx, 0)

        def zr(i, _):
            b = i // (BLK * DH // L)
            rem = i % (BLK * DH // L)
            r = rem // (DH // L)
            l = rem % (DH // L)
            rbuf[b, r, pl.ds(l * L, L)] = z16
            return 0

        lax.fori_loop(0, 2 * BLK * DH // L, zr, 0)

        row = s * FR
        pltpu.sync_copy(rbuf.at[0], acc.at[pl.ds(row, BLK)])
        pltpu.sync_copy(rbuf.at[1], acc.at[pl.ds(row + BLK, BLK)])
        pltpu.sync_copy(rbuf.at[0, pl.ds(0, FR - 2 * BLK)],
                        acc.at[pl.ds(row + 2 * BLK, FR - 2 * BLK)])
        plsc.subcore_barrier()

        def start(j, buf, sem):
            return pltpu.async_copy(
                tbl_hbm.at[srcx.at[pl.ds(j * BLK, BLK)]], rbuf.at[buf], sem)

        start(0, 0, gsem0)

        def pipe(i, _):
            j0 = 2 * i
            j1 = 2 * i + 1
            pltpu.make_async_copy(
                tbl_hbm.at[srcx.at[pl.ds(j0 * BLK, BLK)]], rbuf.at[0], gsem0
            ).wait()
            start(j1, 1, gsem1)
            pltpu.sync_copy(rbuf.at[0], acc.at[idx2d.at[j0]], add=True)
            pltpu.make_async_copy(
                tbl_hbm.at[srcx.at[pl.ds(j1 * BLK, BLK)]], rbuf.at[1], gsem1
            ).wait()

            @pl.when(i + 1 < npairs)
            def _():
                start(j1 + 1, 0, gsem0)

            pltpu.sync_copy(rbuf.at[1], acc.at[idx2d.at[j1]], add=True)
            return 0

        lax.fori_loop(0, npairs, pipe, 0)

        plsc.subcore_barrier()
        col = h * DH
        pltpu.sync_copy(acc.at[pl.ds(row, BLK)], rbuf.at[0])
        cp0 = pltpu.async_copy(
            rbuf.at[0], out_hbm.at[c, pl.ds(row, BLK), pl.ds(col, DH)], gsem0)
        pltpu.sync_copy(acc.at[pl.ds(row + BLK, BLK)], rbuf.at[1])
        cp1 = pltpu.async_copy(
            rbuf.at[1], out_hbm.at[c, pl.ds(row + BLK, BLK), pl.ds(col, DH)],
            gsem1)
        cp0.wait()
        pltpu.sync_copy(acc.at[pl.ds(row + 2 * BLK, FR - 2 * BLK)],
                        rbuf.at[0, pl.ds(0, FR - 2 * BLK)])
        cp1.wait()
        pltpu.sync_copy(
            rbuf.at[0, pl.ds(0, FR - 2 * BLK)],
            out_hbm.at[c, pl.ds(row + 2 * BLK, FR - 2 * BLK), pl.ds(col, DH)])


_agg_call = functools.partial(
    pl.kernel,
    out_type=jax.ShapeDtypeStruct((NC, SLABR, D), jnp.float32),
    mesh=_mesh,
    scratch_types=[
        pltpu.VMEM((ECP,), jnp.int32),
        pltpu.VMEM((ECP,), jnp.int32),
        pltpu.VMEM((ECP,), jnp.int32),
        pltpu.VMEM((NB, BLK), jnp.int32),
        pltpu.VMEM((2, BLK, DH), jnp.float32),
        pltpu.VMEM_SHARED((SLABR, DH), jnp.float32),
        pltpu.SemaphoreType.DMA,
        pltpu.SemaphoreType.DMA,
    ],
)(_agg_body)


def _prep_body(feat_ref, deg_ref, hs1_ref, norm_ref, norm2_ref):
    d = jnp.maximum(deg_ref[...][0], 1.0)
    nr = lax.rsqrt(d)
    hs1_ref[...] = (feat_ref[...] * nr)[None]
    norm_ref[...] = nr[None]
    norm2_ref[...] = (nr * nr)[None]


_GB = 625


def _prep_call(feat, deg_slab3):
    return pl.pallas_call(
        _prep_body,
        grid=(2 * _GB,),
        in_specs=[
            pl.BlockSpec((8, D), lambda i: (i, 0)),
            pl.BlockSpec((1, 8, 1), lambda i: (i // _GB, i % _GB, 0)),
        ],
        out_specs=[
            pl.BlockSpec((1, 8, D), lambda i: (i // _GB, i % _GB, 0)),
            pl.BlockSpec((1, 8, 1), lambda i: (i // _GB, i % _GB, 0)),
            pl.BlockSpec((1, 8, 1), lambda i: (i // _GB, i % _GB, 0)),
        ],
        out_shape=[
            jax.ShapeDtypeStruct((NC, SLABR, D), jnp.float32),
            jax.ShapeDtypeStruct((NC, SLABR, 1), jnp.float32),
            jax.ShapeDtypeStruct((NC, SLABR, 1), jnp.float32),
        ],
    )(feat, deg_slab3)


def _scale_body(x_ref, n_ref, o_ref):
    o_ref[...] = x_ref[...] * n_ref[...]


def _scale_call(x, n):
    return pl.pallas_call(
        _scale_body,
        grid=(NC, SLABR // 256),
        in_specs=[
            pl.BlockSpec((1, 256, D), lambda a, b: (a, b, 0)),
            pl.BlockSpec((1, 256, 1), lambda a, b: (a, b, 0)),
        ],
        out_specs=pl.BlockSpec((1, 256, D), lambda a, b: (a, b, 0)),
        out_shape=jax.ShapeDtypeStruct((NC, SLABR, D), jnp.float32),
    )(x, n)


def kernel(feat, edge_index):
    src = edge_index[0].astype(jnp.int32)
    dst = edge_index[1].astype(jnp.int32)

    deg_slab = _deg_call(dst).reshape(NC, SLABR, 1)
    hs1_slab, norm_slab, norm2_slab = _prep_call(feat, deg_slab)
    acc1 = _agg_call(src, dst, hs1_slab.reshape(2 * TBL, D // 2))
    hs2_slab = _scale_call(acc1, norm2_slab)
    acc2 = _agg_call(src, dst, hs2_slab.reshape(2 * TBL, D // 2))
    out_slab = _scale_call(acc2, norm_slab)
    return jnp.concatenate([out_slab[0, :HALF], out_slab[1, :HALF]], axis=0)


# speedup vs baseline: 1.4884x; 1.4884x over previous
import functools

import jax
import jax.numpy as jnp
from jax import lax
from jax.experimental import pallas as pl
from jax.experimental.pallas import tpu as pltpu
from jax.experimental.pallas import tpu_sc as plsc

N = 10000
E = 160000
D = 256
NC = 2
NS = 16
L = 16

NP = 10240
FR = NP // NS
DUMP = N
EC = E // (NC * NS)
BLK = 128
NB = 40
ECP = NB * BLK
PAD0 = (EC // L) * L
DH = D // 2

_mesh = plsc.VectorSubcoreMesh(
    core_axis_name="c", subcore_axis_name="s", num_cores=NC, num_subcores=NS
)


def _load_edge_chunk(ehbm, ebuf, t, pad_value):
    pad = jnp.full((L,), pad_value, jnp.int32)

    def body(i, _):
        ebuf[pl.ds(PAD0 + i * L, L)] = pad
        return 0

    lax.fori_loop(0, (ECP - PAD0) // L, body, 0)
    pltpu.sync_copy(ehbm.at[pl.ds(t * EC, EC)], ebuf.at[pl.ds(0, EC)])


def _build_dst_idx(dstbuf, idx2d, lanes_mod):
    lane = lax.iota(jnp.int32, L)

    def body(i, _):
        j = i // (BLK // L)
        l = i % (BLK // L)
        v = dstbuf[pl.ds(i * L, L)]
        ok = (v >= 0) & (v < N)
        dump = DUMP + (i % lanes_mod) * L + lane
        idx2d[j, pl.ds(l * L, L)] = jnp.where(ok, v, dump)
        return 0

    lax.fori_loop(0, ECP // L, body, 0)


def _deg_body(dst_hbm, deg_hbm, dstbuf, idx2d, ones, zb, acc):
    c = lax.axis_index("c")
    s = lax.axis_index("s")
    t = c * NS + s

    z16 = jnp.zeros((L,), jnp.float32)
    o16 = jnp.ones((L,), jnp.float32)

    def fill(i, _):
        zb[pl.ds(i * L, L)] = z16
        return 0

    lax.fori_loop(0, FR // L, fill, 0)

    def fill1(i, _):
        ones[pl.ds(i * L, L)] = o16
        return 0

    lax.fori_loop(0, BLK // L, fill1, 0)

    _load_edge_chunk(dst_hbm, dstbuf, t, -1)
    _build_dst_idx(dstbuf, idx2d, 15)

    pltpu.sync_copy(zb, acc.at[pl.ds(s * FR, FR)])
    plsc.subcore_barrier()

    def scat(j, _):
        pltpu.sync_copy(ones, acc.at[idx2d.at[j]], add=True)
        return 0

    lax.fori_loop(0, NB, scat, 0)
    plsc.subcore_barrier()
    pltpu.sync_copy(acc.at[pl.ds(s * FR, FR)], zb)
    pltpu.sync_copy(zb, deg_hbm.at[c, pl.ds(s * FR, FR)])


_deg_call = functools.partial(
    pl.kernel,
    out_type=jax.ShapeDtypeStruct((NC, NP), jnp.float32),
    mesh=_mesh,
    scratch_types=[
        pltpu.VMEM((ECP,), jnp.int32),
        pltpu.VMEM((NB, BLK), jnp.int32),
        pltpu.VMEM((BLK,), jnp.float32),
        pltpu.VMEM((FR,), jnp.float32),
        pltpu.VMEM_SHARED((NP,), jnp.float32),
    ],
)(_deg_body)


def _agg_body(src_hbm, dst_hbm, tbl_hbm, out_hbm, srcb, dstb,
              idx2d, rbuf, acc, gsem0, gsem1):
    c = lax.axis_index("c")
    s = lax.axis_index("s")
    t = c * NS + s

    z16 = jnp.zeros((L,), jnp.float32)

    _load_edge_chunk(src_hbm, srcb, t, 0)
    _load_edge_chunk(dst_hbm, dstb, t, -1)
    _build_dst_idx(dstb, idx2d, 15)

    npairs = NB // 2

    for h in range(2):
        def half_idx(i, _):
            v = srcb[pl.ds(i * L, L)]
            srcb[pl.ds(i * L, L)] = (2 * v) if h == 0 else (v + 1)
            return 0

        lax.fori_loop(0, ECP // L, half_idx, 0)

        def zr(i, _):
            b = i // (BLK * DH // L)
            rem = i % (BLK * DH // L)
            r = rem // (DH // L)
            l = rem % (DH // L)
            rbuf[b, r, pl.ds(l * L, L)] = z16
            return 0

        lax.fori_loop(0, 2 * BLK * DH // L, zr, 0)

        row = s * FR
        for q in range(FR // BLK):
            pltpu.sync_copy(rbuf.at[q % 2], acc.at[pl.ds(row + q * BLK, BLK)])
        plsc.subcore_barrier()

        def start(j, buf, sem):
            return pltpu.async_copy(
                tbl_hbm.at[srcb.at[pl.ds(j * BLK, BLK)]], rbuf.at[buf], sem)

        start(0, 0, gsem0)

        def pipe(i, _):
            j0 = 2 * i
            j1 = 2 * i + 1
            pltpu.make_async_copy(
                tbl_hbm.at[srcb.at[pl.ds(j0 * BLK, BLK)]], rbuf.at[0], gsem0
            ).wait()
            start(j1, 1, gsem1)
            pltpu.sync_copy(rbuf.at[0], acc.at[idx2d.at[j0]], add=True)
            pltpu.make_async_copy(
                tbl_hbm.at[srcb.at[pl.ds(j1 * BLK, BLK)]], rbuf.at[1], gsem1
            ).wait()

            @pl.when(i + 1 < npairs)
            def _():
                start(j1 + 1, 0, gsem0)

            pltpu.sync_copy(rbuf.at[1], acc.at[idx2d.at[j1]], add=True)
            return 0

        lax.fori_loop(0, npairs, pipe, 0)

        plsc.subcore_barrier()
        col = h * DH
        cps = []
        for q in range(FR // BLK):
            if q >= 2:
                cps[q - 2].wait()
            pltpu.sync_copy(acc.at[pl.ds(row + q * BLK, BLK)], rbuf.at[q % 2])
            cps.append(pltpu.async_copy(
                rbuf.at[q % 2],
                out_hbm.at[c, pl.ds(row + q * BLK, BLK), pl.ds(col, DH)],
                gsem0 if q % 2 == 0 else gsem1))
        cps[-2].wait()
        cps[-1].wait()


_agg_call = functools.partial(
    pl.kernel,
    out_type=jax.ShapeDtypeStruct((NC, NP, D), jnp.float32),
    mesh=_mesh,
    scratch_types=[
        pltpu.VMEM((ECP,), jnp.int32),
        pltpu.VMEM((ECP,), jnp.int32),
        pltpu.VMEM((NB, BLK), jnp.int32),
        pltpu.VMEM((2, BLK, DH), jnp.float32),
        pltpu.VMEM_SHARED((NP, DH), jnp.float32),
        pltpu.SemaphoreType.DMA,
        pltpu.SemaphoreType.DMA,
    ],
)(_agg_body)


def _prep_body(feat_ref, deg0_ref, deg1_ref, hs1_ref, norm_ref, norm2_ref):
    d = jnp.maximum(deg0_ref[...][0] + deg1_ref[...][0], 1.0)
    nr = lax.rsqrt(d)
    hs1_ref[...] = feat_ref[...] * nr
    norm_ref[...] = nr
    norm2_ref[...] = nr * nr


_GB = N // 8


def _prep_call(feat, deg3):
    return pl.pallas_call(
        _prep_body,
        grid=(_GB,),
        in_specs=[
            pl.BlockSpec((8, D), lambda i: (i, 0)),
            pl.BlockSpec((1, 8, 1), lambda i: (0, i, 0)),
            pl.BlockSpec((1, 8, 1), lambda i: (1, i, 0)),
        ],
        out_specs=[
            pl.BlockSpec((8, D), lambda i: (i, 0)),
            pl.BlockSpec((8, 1), lambda i: (i, 0)),
            pl.BlockSpec((8, 1), lambda i: (i, 0)),
        ],
        out_shape=[
            jax.ShapeDtypeStruct((NP, D), jnp.float32),
            jax.ShapeDtypeStruct((NP, 1), jnp.float32),
            jax.ShapeDtypeStruct((NP, 1), jnp.float32),
        ],
    )(feat, deg3, deg3)


def _scale_body(x0_ref, x1_ref, n_ref, o_ref):
    o_ref[...] = (x0_ref[...][0] + x1_ref[...][0]) * n_ref[...]


def _scale_call(parts, n):
    return pl.pallas_call(
        _scale_body,
        grid=(NP // 256,),
        in_specs=[
            pl.BlockSpec((1, 256, D), lambda b: (0, b, 0)),
            pl.BlockSpec((1, 256, D), lambda b: (1, b, 0)),
            pl.BlockSpec((256, 1), lambda b: (b, 0)),
        ],
        out_specs=pl.BlockSpec((256, D), lambda b: (b, 0)),
        out_shape=jax.ShapeDtypeStruct((NP, D), jnp.float32),
    )(parts, parts, n)


def kernel(feat, edge_index):
    src = edge_index[0].astype(jnp.int32)
    dst = edge_index[1].astype(jnp.int32)

    deg3 = _deg_call(dst).reshape(NC, NP, 1)
    hs1, norm, norm2 = _prep_call(feat, deg3)
    acc1 = _agg_call(src, dst, hs1.reshape(2 * NP, DH))
    hs2 = _scale_call(acc1, norm2)
    acc2 = _agg_call(src, dst, hs2.reshape(2 * NP, DH))
    out = _scale_call(acc2, norm)
    return out[:N]


# speedup vs baseline: 2.3214x; 1.5597x over previous
import functools

import jax
import jax.numpy as jnp
from jax import lax
from jax.experimental import pallas as pl
from jax.experimental.pallas import tpu as pltpu
from jax.experimental.pallas import tpu_sc as plsc

N = 10000
E = 160000
D = 256
NC = 2
NS = 16
L = 16

NP = 10240
FR = NP // NS
DUMP = N
EC = E // (NC * NS)
BLK = 128
NB = 40
ECP = NB * BLK
PAD0 = (EC // L) * L
DH = D // 2

_mesh = plsc.VectorSubcoreMesh(
    core_axis_name="c", subcore_axis_name="s", num_cores=NC, num_subcores=NS
)


def _load_edge_chunk(ehbm, ebuf, t, pad_value):
    pad = jnp.full((L,), pad_value, jnp.int32)

    def body(i, _):
        ebuf[pl.ds(PAD0 + i * L, L)] = pad
        return 0

    lax.fori_loop(0, (ECP - PAD0) // L, body, 0)
    pltpu.sync_copy(ehbm.at[pl.ds(t * EC, EC)], ebuf.at[pl.ds(0, EC)])


def _build_dst_idx(dstbuf, idx2d, lanes_mod):
    lane = lax.iota(jnp.int32, L)

    def body(i, _):
        j = i // (BLK // L)
        l = i % (BLK // L)
        v = dstbuf[pl.ds(i * L, L)]
        ok = (v >= 0) & (v < N)
        dump = DUMP + (i % lanes_mod) * L + lane
        idx2d[j, pl.ds(l * L, L)] = jnp.where(ok, v, dump)
        return 0

    lax.fori_loop(0, ECP // L, body, 0)


def _deg_body(dst_hbm, deg_hbm, dstbuf, idx2d, ones, zb, acc):
    c = lax.axis_index("c")
    s = lax.axis_index("s")
    t = c * NS + s

    z16 = jnp.zeros((L,), jnp.float32)
    o16 = jnp.ones((L,), jnp.float32)

    def fill(i, _):
        zb[pl.ds(i * L, L)] = z16
        return 0

    lax.fori_loop(0, FR // L, fill, 0)

    def fill1(i, _):
        ones[pl.ds(i * L, L)] = o16
        return 0

    lax.fori_loop(0, BLK // L, fill1, 0)

    _load_edge_chunk(dst_hbm, dstbuf, t, -1)
    _build_dst_idx(dstbuf, idx2d, 15)

    pltpu.sync_copy(zb, acc.at[pl.ds(s * FR, FR)])
    plsc.subcore_barrier()

    def scat(j, _):
        pltpu.sync_copy(ones, acc.at[idx2d.at[j]], add=True)
        return 0

    lax.fori_loop(0, NB, scat, 0)
    plsc.subcore_barrier()
    pltpu.sync_copy(acc.at[pl.ds(s * FR, FR)], zb)
    pltpu.sync_copy(zb, deg_hbm.at[c, pl.ds(s * FR, FR)])


_deg_call = functools.partial(
    pl.kernel,
    out_type=jax.ShapeDtypeStruct((NC, NP), jnp.float32),
    mesh=_mesh,
    scratch_types=[
        pltpu.VMEM((ECP,), jnp.int32),
        pltpu.VMEM((NB, BLK), jnp.int32),
        pltpu.VMEM((BLK,), jnp.float32),
        pltpu.VMEM((FR,), jnp.float32),
        pltpu.VMEM_SHARED((NP,), jnp.float32),
    ],
)(_deg_body)


def _agg_body(src_hbm, dst_hbm, tbl_hbm, out_hbm, srcb, dstb,
              idx2d, rbuf, acc, gsem0, gsem1):
    c = lax.axis_index("c")
    s = lax.axis_index("s")
    t = c * NS + s

    z16 = jnp.zeros((L,), jnp.float32)

    _load_edge_chunk(src_hbm, srcb, t, 0)
    _load_edge_chunk(dst_hbm, dstb, t, -1)
    _build_dst_idx(dstb, idx2d, 15)

    npairs = NB // 2

    for h in range(2):
        def half_idx(i, _):
            v = srcb[pl.ds(i * L, L)]
            srcb[pl.ds(i * L, L)] = (2 * v) if h == 0 else (v + 1)
            return 0

        lax.fori_loop(0, ECP // L, half_idx, 0)

        def zr(i, _):
            b = i // (BLK * DH // L)
            rem = i % (BLK * DH // L)
            r = rem // (DH // L)
            l = rem % (DH // L)
            rbuf[b, r, pl.ds(l * L, L)] = z16
            return 0

        lax.fori_loop(0, 2 * BLK * DH // L, zr, 0)

        row = s * FR
        for q in range(FR // BLK):
            pltpu.sync_copy(rbuf.at[q % 2], acc.at[pl.ds(row + q * BLK, BLK)])
        plsc.subcore_barrier()

        def start(j, buf, sem):
            return pltpu.async_copy(
                tbl_hbm.at[srcb.at[pl.ds(j * BLK, BLK)]], rbuf.at[buf], sem)

        start(0, 0, gsem0)

        def pipe(i, _):
            j0 = 2 * i
            j1 = 2 * i + 1
            pltpu.make_async_copy(
                tbl_hbm.at[srcb.at[pl.ds(j0 * BLK, BLK)]], rbuf.at[0], gsem0
            ).wait()
            start(j1, 1, gsem1)
            pltpu.sync_copy(rbuf.at[0], acc.at[idx2d.at[j0]], add=True)
            pltpu.make_async_copy(
                tbl_hbm.at[srcb.at[pl.ds(j1 * BLK, BLK)]], rbuf.at[1], gsem1
            ).wait()

            @pl.when(i + 1 < npairs)
            def _():
                start(j1 + 1, 0, gsem0)

            pltpu.sync_copy(rbuf.at[1], acc.at[idx2d.at[j1]], add=True)
            return 0

        lax.fori_loop(0, npairs, pipe, 0)

        plsc.subcore_barrier()
        col = h * DH
        cps = []
        for q in range(FR // BLK):
            if q >= 2:
                cps[q - 2].wait()
            pltpu.sync_copy(acc.at[pl.ds(row + q * BLK, BLK)], rbuf.at[q % 2])
            cps.append(pltpu.async_copy(
                rbuf.at[q % 2],
                out_hbm.at[c, pl.ds(row + q * BLK, BLK), pl.ds(col, DH)],
                gsem0 if q % 2 == 0 else gsem1))
        cps[-2].wait()
        cps[-1].wait()


_agg_call = functools.partial(
    pl.kernel,
    out_type=jax.ShapeDtypeStruct((NC, NP, D), jnp.float32),
    mesh=_mesh,
    scratch_types=[
        pltpu.VMEM((ECP,), jnp.int32),
        pltpu.VMEM((ECP,), jnp.int32),
        pltpu.VMEM((NB, BLK), jnp.int32),
        pltpu.VMEM((2, BLK, DH), jnp.float32),
        pltpu.VMEM_SHARED((NP, DH), jnp.float32),
        pltpu.SemaphoreType.DMA,
        pltpu.SemaphoreType.DMA,
    ],
)(_agg_body)


def _prep_body(feat_ref, deg0_ref, deg1_ref, hs1_ref, norm_ref, norm2_ref):
    d = jnp.maximum(deg0_ref[...][0] + deg1_ref[...][0], 1.0)
    nr = lax.rsqrt(d)
    hs1_ref[...] = feat_ref[...] * nr
    norm_ref[...] = nr
    norm2_ref[...] = nr * nr


def _prep_call(featp, deg3):
    return pl.pallas_call(
        _prep_body,
        grid=(NP // 256,),
        in_specs=[
            pl.BlockSpec((256, D), lambda i: (i, 0)),
            pl.BlockSpec((1, 256, 1), lambda i: (0, i, 0)),
            pl.BlockSpec((1, 256, 1), lambda i: (1, i, 0)),
        ],
        out_specs=[
            pl.BlockSpec((256, D), lambda i: (i, 0)),
            pl.BlockSpec((256, 1), lambda i: (i, 0)),
            pl.BlockSpec((256, 1), lambda i: (i, 0)),
        ],
        out_shape=[
            jax.ShapeDtypeStruct((NP, D), jnp.float32),
            jax.ShapeDtypeStruct((NP, 1), jnp.float32),
            jax.ShapeDtypeStruct((NP, 1), jnp.float32),
        ],
    )(featp, deg3, deg3)


def _scale_body(x0_ref, x1_ref, n_ref, o_ref):
    o_ref[...] = (x0_ref[...][0] + x1_ref[...][0]) * n_ref[...]


def _scale_call(parts, n):
    return pl.pallas_call(
        _scale_body,
        grid=(NP // 256,),
        in_specs=[
            pl.BlockSpec((1, 256, D), lambda b: (0, b, 0)),
            pl.BlockSpec((1, 256, D), lambda b: (1, b, 0)),
            pl.BlockSpec((256, 1), lambda b: (b, 0)),
        ],
        out_specs=pl.BlockSpec((256, D), lambda b: (b, 0)),
        out_shape=jax.ShapeDtypeStruct((NP, D), jnp.float32),
    )(parts, parts, n)


def kernel(feat, edge_index):
    src = edge_index[0].astype(jnp.int32)
    dst = edge_index[1].astype(jnp.int32)

    featp = jnp.zeros((NP, D), jnp.float32).at[:N].set(feat)
    deg3 = _deg_call(dst).reshape(NC, NP, 1)
    hs1, norm, norm2 = _prep_call(featp, deg3)
    acc1 = _agg_call(src, dst, hs1.reshape(2 * NP, DH))
    hs2 = _scale_call(acc1, norm2)
    acc2 = _agg_call(src, dst, hs2.reshape(2 * NP, DH))
    out = _scale_call(acc2, norm)
    return out[:N]


# speedup vs baseline: 2.3542x; 1.0141x over previous
import functools

import jax
import jax.numpy as jnp
from jax import lax
from jax.experimental import pallas as pl
from jax.experimental.pallas import tpu as pltpu
from jax.experimental.pallas import tpu_sc as plsc

N = 10000
E = 160000
D = 256
NC = 2
NS = 16
L = 16

NP = 10240
FR = NP // NS
DUMP = N
EC = E // (NC * NS)
BLK = 64
NB = 80
NBUF = 2
ECP = NB * BLK
PAD0 = (EC // L) * L
DH = D // 2

_mesh = plsc.VectorSubcoreMesh(
    core_axis_name="c", subcore_axis_name="s", num_cores=NC, num_subcores=NS
)


def _load_edge_chunk(ehbm, ebuf, t, pad_value):
    pad = jnp.full((L,), pad_value, jnp.int32)

    def body(i, _):
        ebuf[pl.ds(PAD0 + i * L, L)] = pad
        return 0

    lax.fori_loop(0, (ECP - PAD0) // L, body, 0)
    pltpu.sync_copy(ehbm.at[pl.ds(t * EC, EC)], ebuf.at[pl.ds(0, EC)])


def _build_dst_idx(dstbuf, idx2d, lanes_mod):
    lane = lax.iota(jnp.int32, L)

    def body(i, _):
        j = i // (BLK // L)
        l = i % (BLK // L)
        v = dstbuf[pl.ds(i * L, L)]
        ok = (v >= 0) & (v < N)
        dump = DUMP + (i % lanes_mod) * L + lane
        idx2d[j, pl.ds(l * L, L)] = jnp.where(ok, v, dump)
        return 0

    lax.fori_loop(0, ECP // L, body, 0)


def _deg_body(dst_hbm, deg_hbm, dstbuf, idx2d, ones, zb, acc):
    c = lax.axis_index("c")
    s = lax.axis_index("s")
    t = c * NS + s

    z16 = jnp.zeros((L,), jnp.float32)
    o16 = jnp.ones((L,), jnp.float32)

    def fill(i, _):
        zb[pl.ds(i * L, L)] = z16
        return 0

    lax.fori_loop(0, FR // L, fill, 0)

    def fill1(i, _):
        ones[pl.ds(i * L, L)] = o16
        return 0

    lax.fori_loop(0, BLK // L, fill1, 0)

    _load_edge_chunk(dst_hbm, dstbuf, t, -1)
    _build_dst_idx(dstbuf, idx2d, 15)

    pltpu.sync_copy(zb, acc.at[pl.ds(s * FR, FR)])
    plsc.subcore_barrier()

    def scat(j, _):
        pltpu.sync_copy(ones, acc.at[idx2d.at[j]], add=True)
        return 0

    lax.fori_loop(0, NB, scat, 0)
    plsc.subcore_barrier()
    pltpu.sync_copy(acc.at[pl.ds(s * FR, FR)], zb)
    pltpu.sync_copy(zb, deg_hbm.at[c, pl.ds(s * FR, FR)])


_deg_call = functools.partial(
    pl.kernel,
    out_type=jax.ShapeDtypeStruct((NC, NP), jnp.float32),
    mesh=_mesh,
    scratch_types=[
        pltpu.VMEM((ECP,), jnp.int32),
        pltpu.VMEM((NB, BLK), jnp.int32),
        pltpu.VMEM((BLK,), jnp.float32),
        pltpu.VMEM((FR,), jnp.float32),
        pltpu.VMEM_SHARED((NP,), jnp.float32),
    ],
)(_deg_body)


def _agg_body(src_hbm, dst_hbm, tbl_hbm, out_hbm, srcb, dstb,
              idx2d, rbuf, acc, *gsems):
    c = lax.axis_index("c")
    s = lax.axis_index("s")
    t = c * NS + s

    z16 = jnp.zeros((L,), jnp.float32)

    _load_edge_chunk(src_hbm, srcb, t, 0)
    _load_edge_chunk(dst_hbm, dstb, t, -1)
    _build_dst_idx(dstb, idx2d, 15)

    ngrp = NB // NBUF

    for h in range(2):
        def half_idx(i, _):
            v = srcb[pl.ds(i * L, L)]
            srcb[pl.ds(i * L, L)] = (2 * v) if h == 0 else (v + 1)
            return 0

        lax.fori_loop(0, ECP // L, half_idx, 0)

        def zr(i, _):
            b = i // (BLK * DH // L)
            rem = i % (BLK * DH // L)
            r = rem // (DH // L)
            l = rem % (DH // L)
            rbuf[b, r, pl.ds(l * L, L)] = z16
            return 0

        lax.fori_loop(0, NBUF * BLK * DH // L, zr, 0)

        row = s * FR
        for q in range(FR // BLK):
            pltpu.sync_copy(rbuf.at[q % NBUF],
                            acc.at[pl.ds(row + q * BLK, BLK)])
        plsc.subcore_barrier()

        def start(j, buf, sem):
            return pltpu.async_copy(
                tbl_hbm.at[srcb.at[pl.ds(j * BLK, BLK)]], rbuf.at[buf], sem)

        for k in range(NBUF):
            start(k, k, gsems[k])

        def grp(i, _):
            for k in range(NBUF):
                j = NBUF * i + k
                pltpu.make_async_copy(
                    tbl_hbm.at[srcb.at[pl.ds(j * BLK, BLK)]], rbuf.at[k],
                    gsems[k]).wait()
                pltpu.sync_copy(rbuf.at[k], acc.at[idx2d.at[j]], add=True)

                @pl.when(i + 1 < ngrp)
                def _():
                    start(j + NBUF, k, gsems[k])

            return 0

        lax.fori_loop(0, ngrp, grp, 0)

        plsc.subcore_barrier()
        col = h * DH
        cps = []
        for q in range(FR // BLK):
            if q >= NBUF:
                cps[q - NBUF].wait()
            pltpu.sync_copy(acc.at[pl.ds(row + q * BLK, BLK)],
                            rbuf.at[q % NBUF])
            cps.append(pltpu.async_copy(
                rbuf.at[q % NBUF],
                out_hbm.at[c, pl.ds(row + q * BLK, BLK), pl.ds(col, DH)],
                gsems[q % NBUF]))
        for cp in cps[-NBUF:]:
            cp.wait()


_agg_call = functools.partial(
    pl.kernel,
    out_type=jax.ShapeDtypeStruct((NC, NP, D), jnp.float32),
    mesh=_mesh,
    scratch_types=[
        pltpu.VMEM((ECP,), jnp.int32),
        pltpu.VMEM((ECP,), jnp.int32),
        pltpu.VMEM((NB, BLK), jnp.int32),
        pltpu.VMEM((NBUF, BLK, DH), jnp.float32),
        pltpu.VMEM_SHARED((NP, DH), jnp.float32),
    ] + [pltpu.SemaphoreType.DMA] * NBUF,
)(_agg_body)


def _prep_body(feat_ref, deg0_ref, deg1_ref, hs1_ref, norm_ref, norm2_ref):
    d = jnp.maximum(deg0_ref[...][0] + deg1_ref[...][0], 1.0)
    nr = lax.rsqrt(d)
    hs1_ref[...] = feat_ref[...] * nr
    norm_ref[...] = nr
    norm2_ref[...] = nr * nr


def _prep_call(featp, deg3):
    return pl.pallas_call(
        _prep_body,
        grid=(NP // 256,),
        in_specs=[
            pl.BlockSpec((256, D), lambda i: (i, 0)),
            pl.BlockSpec((1, 256, 1), lambda i: (0, i, 0)),
            pl.BlockSpec((1, 256, 1), lambda i: (1, i, 0)),
        ],
        out_specs=[
            pl.BlockSpec((256, D), lambda i: (i, 0)),
            pl.BlockSpec((256, 1), lambda i: (i, 0)),
            pl.BlockSpec((256, 1), lambda i: (i, 0)),
        ],
        out_shape=[
            jax.ShapeDtypeStruct((NP, D), jnp.float32),
            jax.ShapeDtypeStruct((NP, 1), jnp.float32),
            jax.ShapeDtypeStruct((NP, 1), jnp.float32),
        ],
    )(featp, deg3, deg3)


def _scale_body(x0_ref, x1_ref, n_ref, o_ref):
    o_ref[...] = (x0_ref[...][0] + x1_ref[...][0]) * n_ref[...]


def _scale_call(parts, n):
    return pl.pallas_call(
        _scale_body,
        grid=(NP // 256,),
        in_specs=[
            pl.BlockSpec((1, 256, D), lambda b: (0, b, 0)),
            pl.BlockSpec((1, 256, D), lambda b: (1, b, 0)),
            pl.BlockSpec((256, 1), lambda b: (b, 0)),
        ],
        out_specs=pl.BlockSpec((256, D), lambda b: (b, 0)),
        out_shape=jax.ShapeDtypeStruct((NP, D), jnp.float32),
    )(parts, parts, n)


def kernel(feat, edge_index):
    src = edge_index[0].astype(jnp.int32)
    dst = edge_index[1].astype(jnp.int32)

    featp = jnp.zeros((NP, D), jnp.float32).at[:N].set(feat)
    deg3 = _deg_call(dst).reshape(NC, NP, 1)
    hs1, norm, norm2 = _prep_call(featp, deg3)
    acc1 = _agg_call(src, dst, hs1.reshape(2 * NP, DH))
    hs2 = _scale_call(acc1, norm2)
    acc2 = _agg_call(src, dst, hs2.reshape(2 * NP, DH))
    out = _scale_call(acc2, norm)
    return out[:N]


# speedup vs baseline: 2.4204x; 1.0281x over previous
import functools

import jax
import jax.numpy as jnp
from jax import lax
from jax.experimental import pallas as pl
from jax.experimental.pallas import tpu as pltpu
from jax.experimental.pallas import tpu_sc as plsc

N = 10000
E = 160000
D = 256
NC = 2
NS = 16
L = 16

NP = 10240
FR = NP // NS
DUMP = N
EC = E // (NC * NS)
BLK = 128
NB = 40
NBUF = 2
ECP = NB * BLK
PAD0 = (EC // L) * L
DH = D // 2

_mesh = plsc.VectorSubcoreMesh(
    core_axis_name="c", subcore_axis_name="s", num_cores=NC, num_subcores=NS
)


def _load_edge_chunk(ehbm, ebuf, t, pad_value):
    pad = jnp.full((L,), pad_value, jnp.int32)

    def body(i, _):
        ebuf[pl.ds(PAD0 + i * L, L)] = pad
        return 0

    lax.fori_loop(0, (ECP - PAD0) // L, body, 0)
    pltpu.sync_copy(ehbm.at[pl.ds(t * EC, EC)], ebuf.at[pl.ds(0, EC)])


def _build_dst_idx(dstbuf, idx2d, lanes_mod):
    lane = lax.iota(jnp.int32, L)

    def body(i, _):
        j = i // (BLK // L)
        l = i % (BLK // L)
        v = dstbuf[pl.ds(i * L, L)]
        ok = (v >= 0) & (v < N)
        dump = DUMP + (i % lanes_mod) * L + lane
        idx2d[j, pl.ds(l * L, L)] = jnp.where(ok, v, dump)
        return 0

    lax.fori_loop(0, ECP // L, body, 0)


def _deg_body(dst_hbm, deg_hbm, dstbuf, idx2d, ones, zb, acc):
    c = lax.axis_index("c")
    s = lax.axis_index("s")
    t = c * NS + s

    z16 = jnp.zeros((L,), jnp.float32)
    o16 = jnp.ones((L,), jnp.float32)

    def fill(i, _):
        zb[pl.ds(i * L, L)] = z16
        return 0

    lax.fori_loop(0, FR // L, fill, 0)

    def fill1(i, _):
        ones[pl.ds(i * L, L)] = o16
        return 0

    lax.fori_loop(0, BLK // L, fill1, 0)

    _load_edge_chunk(dst_hbm, dstbuf, t, -1)
    _build_dst_idx(dstbuf, idx2d, 15)

    pltpu.sync_copy(zb, acc.at[pl.ds(s * FR, FR)])
    plsc.subcore_barrier()

    def scat(j, _):
        pltpu.sync_copy(ones, acc.at[idx2d.at[j]], add=True)
        return 0

    lax.fori_loop(0, NB, scat, 0)
    plsc.subcore_barrier()
    pltpu.sync_copy(acc.at[pl.ds(s * FR, FR)], zb)
    pltpu.sync_copy(zb, deg_hbm.at[c, pl.ds(s * FR, FR)])


_deg_call = functools.partial(
    pl.kernel,
    out_type=jax.ShapeDtypeStruct((NC, NP), jnp.float32),
    mesh=_mesh,
    scratch_types=[
        pltpu.VMEM((ECP,), jnp.int32),
        pltpu.VMEM((NB, BLK), jnp.int32),
        pltpu.VMEM((BLK,), jnp.float32),
        pltpu.VMEM((FR,), jnp.float32),
        pltpu.VMEM_SHARED((NP,), jnp.float32),
    ],
)(_deg_body)


def _agg_body(src_hbm, dst_hbm, tbl_hbm, out_hbm, srcb,
              idx2d, rbuf, acc, *gsems):
    c = lax.axis_index("c")
    s = lax.axis_index("s")
    t = c * NS + s

    z16 = jnp.zeros((L,), jnp.float32)

    _load_edge_chunk(dst_hbm, srcb, t, -1)
    _build_dst_idx(srcb, idx2d, 15)
    _load_edge_chunk(src_hbm, srcb, t, 0)

    ngrp = NB // NBUF

    for h in range(2):
        def half_idx(i, _):
            v = srcb[pl.ds(i * L, L)]
            srcb[pl.ds(i * L, L)] = (2 * v) if h == 0 else (v + 1)
            return 0

        lax.fori_loop(0, ECP // L, half_idx, 0)

        def zr(i, _):
            b = i // (BLK * DH // L)
            rem = i % (BLK * DH // L)
            r = rem // (DH // L)
            l = rem % (DH // L)
            rbuf[b, r, pl.ds(l * L, L)] = z16
            return 0

        lax.fori_loop(0, NBUF * BLK * DH // L, zr, 0)

        row = s * FR
        for q in range(FR // BLK):
            pltpu.sync_copy(rbuf.at[q % NBUF],
                            acc.at[pl.ds(row + q * BLK, BLK)])
        plsc.subcore_barrier()

        def start(j, buf, sem):
            return pltpu.async_copy(
                tbl_hbm.at[srcb.at[pl.ds(j * BLK, BLK)]], rbuf.at[buf], sem)

        for k in range(NBUF):
            start(k, k, gsems[k])

        def grp(i, _):
            for k in range(NBUF):
                j = NBUF * i + k
                pltpu.make_async_copy(
                    tbl_hbm.at[srcb.at[pl.ds(j * BLK, BLK)]], rbuf.at[k],
                    gsems[k]).wait()
                pltpu.sync_copy(rbuf.at[k], acc.at[idx2d.at[j]], add=True)

                @pl.when(i + 1 < ngrp)
                def _():
                    start(j + NBUF, k, gsems[k])

            return 0

        lax.fori_loop(0, ngrp, grp, 0)

        plsc.subcore_barrier()
        col = h * DH
        cps = []
        for q in range(FR // BLK):
            if q >= NBUF:
                cps[q - NBUF].wait()
            pltpu.sync_copy(acc.at[pl.ds(row + q * BLK, BLK)],
                            rbuf.at[q % NBUF])
            cps.append(pltpu.async_copy(
                rbuf.at[q % NBUF],
                out_hbm.at[c, pl.ds(row + q * BLK, BLK), pl.ds(col, DH)],
                gsems[q % NBUF]))
        for cp in cps[-NBUF:]:
            cp.wait()


_agg_call = functools.partial(
    pl.kernel,
    out_type=jax.ShapeDtypeStruct((NC, NP, D), jnp.float32),
    mesh=_mesh,
    scratch_types=[
        pltpu.VMEM((ECP,), jnp.int32),
        pltpu.VMEM((NB, BLK), jnp.int32),
        pltpu.VMEM((NBUF, BLK, DH), jnp.float32),
        pltpu.VMEM_SHARED((NP, DH), jnp.float32),
    ] + [pltpu.SemaphoreType.DMA] * NBUF,
)(_agg_body)


def _prep_body(feat_ref, deg0_ref, deg1_ref, hs1_ref, norm_ref, norm2_ref):
    d = jnp.maximum(deg0_ref[...][0] + deg1_ref[...][0], 1.0)
    nr = lax.rsqrt(d)
    hs1_ref[...] = feat_ref[...] * nr
    norm_ref[...] = nr
    norm2_ref[...] = nr * nr


def _prep_call(featp, deg3):
    return pl.pallas_call(
        _prep_body,
        grid=(NP // 256,),
        in_specs=[
            pl.BlockSpec((256, D), lambda i: (i, 0)),
            pl.BlockSpec((1, 256, 1), lambda i: (0, i, 0)),
            pl.BlockSpec((1, 256, 1), lambda i: (1, i, 0)),
        ],
        out_specs=[
            pl.BlockSpec((256, D), lambda i: (i, 0)),
            pl.BlockSpec((256, 1), lambda i: (i, 0)),
            pl.BlockSpec((256, 1), lambda i: (i, 0)),
        ],
        out_shape=[
            jax.ShapeDtypeStruct((NP, D), jnp.float32),
            jax.ShapeDtypeStruct((NP, 1), jnp.float32),
            jax.ShapeDtypeStruct((NP, 1), jnp.float32),
        ],
    )(featp, deg3, deg3)


def _scale_body(x0_ref, x1_ref, n_ref, o_ref):
    o_ref[...] = (x0_ref[...][0] + x1_ref[...][0]) * n_ref[...]


def _scale_call(parts, n):
    return pl.pallas_call(
        _scale_body,
        grid=(NP // 256,),
        in_specs=[
            pl.BlockSpec((1, 256, D), lambda b: (0, b, 0)),
            pl.BlockSpec((1, 256, D), lambda b: (1, b, 0)),
            pl.BlockSpec((256, 1), lambda b: (b, 0)),
        ],
        out_specs=pl.BlockSpec((256, D), lambda b: (b, 0)),
        out_shape=jax.ShapeDtypeStruct((NP, D), jnp.float32),
    )(parts, parts, n)


def kernel(feat, edge_index):
    src = edge_index[0].astype(jnp.int32)
    dst = edge_index[1].astype(jnp.int32)

    featp = jnp.zeros((NP, D), jnp.float32).at[:N].set(feat)
    deg3 = _deg_call(dst).reshape(NC, NP, 1)
    hs1, norm, norm2 = _prep_call(featp, deg3)
    acc1 = _agg_call(src, dst, hs1.reshape(2 * NP, DH))
    hs2 = _scale_call(acc1, norm2)
    acc2 = _agg_call(src, dst, hs2.reshape(2 * NP, DH))
    out = _scale_call(acc2, norm)
    return out[:N]


# speedup vs baseline: 2.4637x; 1.0179x over previous
import functools

import jax
import jax.numpy as jnp
from jax import lax
from jax.experimental import pallas as pl
from jax.experimental.pallas import tpu as pltpu
from jax.experimental.pallas import tpu_sc as plsc

N = 10000
E = 160000
D = 256
NC = 2
NS = 16
L = 16

NP = 10240
FR = NP // NS
DUMP = N
EC = E // (NC * NS)
BLK = 64
NB = 80
NBUF = 4
ECP = NB * BLK
PAD0 = (EC // L) * L
DH = D // 2

_mesh = plsc.VectorSubcoreMesh(
    core_axis_name="c", subcore_axis_name="s", num_cores=NC, num_subcores=NS
)


def _load_edge_chunk(ehbm, ebuf, t, pad_value):
    pad = jnp.full((L,), pad_value, jnp.int32)

    def body(i, _):
        ebuf[pl.ds(PAD0 + i * L, L)] = pad
        return 0

    lax.fori_loop(0, (ECP - PAD0) // L, body, 0)
    pltpu.sync_copy(ehbm.at[pl.ds(t * EC, EC)], ebuf.at[pl.ds(0, EC)])


def _build_dst_idx(dstbuf, idx2d, lanes_mod):
    lane = lax.iota(jnp.int32, L)

    def body(i, _):
        j = i // (BLK // L)
        l = i % (BLK // L)
        v = dstbuf[pl.ds(i * L, L)]
        ok = (v >= 0) & (v < N)
        dump = DUMP + (i % lanes_mod) * L + lane
        idx2d[j, pl.ds(l * L, L)] = jnp.where(ok, v, dump)
        return 0

    lax.fori_loop(0, ECP // L, body, 0)


def _deg_body(dst_hbm, deg_hbm, dstbuf, idx2d, ones, zb, acc):
    c = lax.axis_index("c")
    s = lax.axis_index("s")
    t = c * NS + s

    z16 = jnp.zeros((L,), jnp.float32)
    o16 = jnp.ones((L,), jnp.float32)

    def fill(i, _):
        zb[pl.ds(i * L, L)] = z16
        return 0

    lax.fori_loop(0, FR // L, fill, 0)

    def fill1(i, _):
        ones[pl.ds(i * L, L)] = o16
        return 0

    lax.fori_loop(0, BLK // L, fill1, 0)

    _load_edge_chunk(dst_hbm, dstbuf, t, -1)
    _build_dst_idx(dstbuf, idx2d, 15)

    pltpu.sync_copy(zb, acc.at[pl.ds(s * FR, FR)])
    plsc.subcore_barrier()

    def scat(j, _):
        pltpu.sync_copy(ones, acc.at[idx2d.at[j]], add=True)
        return 0

    lax.fori_loop(0, NB, scat, 0)
    plsc.subcore_barrier()
    pltpu.sync_copy(acc.at[pl.ds(s * FR, FR)], zb)
    pltpu.sync_copy(zb, deg_hbm.at[c, pl.ds(s * FR, FR)])


_deg_call = functools.partial(
    pl.kernel,
    out_type=jax.ShapeDtypeStruct((NC, NP), jnp.float32),
    mesh=_mesh,
    scratch_types=[
        pltpu.VMEM((ECP,), jnp.int32),
        pltpu.VMEM((NB, BLK), jnp.int32),
        pltpu.VMEM((BLK,), jnp.float32),
        pltpu.VMEM((FR,), jnp.float32),
        pltpu.VMEM_SHARED((NP,), jnp.float32),
    ],
)(_deg_body)


def _agg_body(src_hbm, dst_hbm, tbl_hbm, out_hbm, srcb,
              idx2d, rbuf, acc, *gsems):
    c = lax.axis_index("c")
    s = lax.axis_index("s")
    t = c * NS + s

    z16 = jnp.zeros((L,), jnp.float32)

    _load_edge_chunk(dst_hbm, srcb, t, -1)
    _build_dst_idx(srcb, idx2d, 15)
    _load_edge_chunk(src_hbm, srcb, t, 0)

    ngrp = NB // NBUF

    for h in range(2):
        def half_idx(i, _):
            v = srcb[pl.ds(i * L, L)]
            srcb[pl.ds(i * L, L)] = (2 * v) if h == 0 else (v + 1)
            return 0

        lax.fori_loop(0, ECP // L, half_idx, 0)

        def zr(i, _):
            b = i // (BLK * DH // L)
            rem = i % (BLK * DH // L)
            r = rem // (DH // L)
            l = rem % (DH // L)
            rbuf[b, r, pl.ds(l * L, L)] = z16
            return 0

        lax.fori_loop(0, NBUF * BLK * DH // L, zr, 0)

        row = s * FR
        for q in range(FR // BLK):
            pltpu.sync_copy(rbuf.at[q % NBUF],
                            acc.at[pl.ds(row + q * BLK, BLK)])
        plsc.subcore_barrier()

        def start(j, buf, sem):
            return pltpu.async_copy(
                tbl_hbm.at[srcb.at[pl.ds(j * BLK, BLK)]], rbuf.at[buf], sem)

        for k in range(NBUF):
            start(k, k, gsems[k])

        def grp(i, _):
            for k in range(NBUF):
                j = NBUF * i + k
                pltpu.make_async_copy(
                    tbl_hbm.at[srcb.at[pl.ds(j * BLK, BLK)]], rbuf.at[k],
                    gsems[k]).wait()
                pltpu.sync_copy(rbuf.at[k], acc.at[idx2d.at[j]], add=True)

                @pl.when(i + 1 < ngrp)
                def _():
                    start(j + NBUF, k, gsems[k])

            return 0

        lax.fori_loop(0, ngrp, grp, 0)

        plsc.subcore_barrier()
        col = h * DH
        cps = []
        for q in range(FR // BLK):
            if q >= NBUF:
                cps[q - NBUF].wait()
            pltpu.sync_copy(acc.at[pl.ds(row + q * BLK, BLK)],
                            rbuf.at[q % NBUF])
            cps.append(pltpu.async_copy(
                rbuf.at[q % NBUF],
                out_hbm.at[c, pl.ds(row + q * BLK, BLK), pl.ds(col, DH)],
                gsems[q % NBUF]))
        for cp in cps[-NBUF:]:
            cp.wait()


_agg_call = functools.partial(
    pl.kernel,
    out_type=jax.ShapeDtypeStruct((NC, NP, D), jnp.float32),
    mesh=_mesh,
    scratch_types=[
        pltpu.VMEM((ECP,), jnp.int32),
        pltpu.VMEM((NB, BLK), jnp.int32),
        pltpu.VMEM((NBUF, BLK, DH), jnp.float32),
        pltpu.VMEM_SHARED((NP, DH), jnp.float32),
    ] + [pltpu.SemaphoreType.DMA] * NBUF,
)(_agg_body)


def _prep_body(feat_ref, deg0_ref, deg1_ref, hs1_ref, norm_ref, norm2_ref):
    d = jnp.maximum(deg0_ref[...][0] + deg1_ref[...][0], 1.0)
    nr = lax.rsqrt(d)
    hs1_ref[...] = feat_ref[...] * nr
    norm_ref[...] = nr
    norm2_ref[...] = nr * nr


def _prep_call(featp, deg3):
    return pl.pallas_call(
        _prep_body,
        grid=(NP // 256,),
        in_specs=[
            pl.BlockSpec((256, D), lambda i: (i, 0)),
            pl.BlockSpec((1, 256, 1), lambda i: (0, i, 0)),
            pl.BlockSpec((1, 256, 1), lambda i: (1, i, 0)),
        ],
        out_specs=[
            pl.BlockSpec((256, D), lambda i: (i, 0)),
            pl.BlockSpec((256, 1), lambda i: (i, 0)),
            pl.BlockSpec((256, 1), lambda i: (i, 0)),
        ],
        out_shape=[
            jax.ShapeDtypeStruct((NP, D), jnp.float32),
            jax.ShapeDtypeStruct((NP, 1), jnp.float32),
            jax.ShapeDtypeStruct((NP, 1), jnp.float32),
        ],
    )(featp, deg3, deg3)


def _scale_body(x0_ref, x1_ref, n_ref, o_ref):
    o_ref[...] = (x0_ref[...][0] + x1_ref[...][0]) * n_ref[...]


def _scale_call(parts, n):
    return pl.pallas_call(
        _scale_body,
        grid=(NP // 256,),
        in_specs=[
            pl.BlockSpec((1, 256, D), lambda b: (0, b, 0)),
            pl.BlockSpec((1, 256, D), lambda b: (1, b, 0)),
            pl.BlockSpec((256, 1), lambda b: (b, 0)),
        ],
        out_specs=pl.BlockSpec((256, D), lambda b: (b, 0)),
        out_shape=jax.ShapeDtypeStruct((NP, D), jnp.float32),
    )(parts, parts, n)


def kernel(feat, edge_index):
    src = edge_index[0].astype(jnp.int32)
    dst = edge_index[1].astype(jnp.int32)

    featp = jnp.zeros((NP, D), jnp.float32).at[:N].set(feat)
    deg3 = _deg_call(dst).reshape(NC, NP, 1)
    hs1, norm, norm2 = _prep_call(featp, deg3)
    acc1 = _agg_call(src, dst, hs1.reshape(2 * NP, DH))
    hs2 = _scale_call(acc1, norm2)
    acc2 = _agg_call(src, dst, hs2.reshape(2 * NP, DH))
    out = _scale_call(acc2, norm)
    return out[:N]
